# baseline (speedup 1.0000x reference)
import functools

import jax
import jax.numpy as jnp
from jax import lax
from jax.experimental import pallas as pl
from jax.experimental.pallas import tpu as pltpu
from jax.experimental.pallas import tpu_sc as plsc

NSEG = 16
D = 2048
TOTAL = 32768
LANES = 16
NCORES = 2
NSUB = 16
NW = NCORES * NSUB
ROWS_PER_W = TOTAL // NW
CHUNK = 16
NCHUNK = ROWS_PER_W // CHUNK
NPAIR = NCHUNK // 2
VPR = D // LANES

_mesh = plsc.VectorSubcoreMesh(
    core_axis_name="c", subcore_axis_name="s",
    num_cores=NCORES, num_subcores=NSUB,
)


@functools.partial(
    pl.kernel,
    out_type=jax.ShapeDtypeStruct((NW, NSEG * D), jnp.float32),
    mesh=_mesh,
    compiler_params=pltpu.CompilerParams(needs_layout_passes=False),
    scratch_types=[
        pltpu.VMEM((LANES,), jnp.int32),
        pltpu.VMEM((CHUNK, D), jnp.float32),
        pltpu.VMEM((CHUNK, D), jnp.float32),
        pltpu.VMEM((NSEG * D,), jnp.float32),
        pltpu.SemaphoreType.DMA,
        pltpu.SemaphoreType.DMA,
    ],
)
def _sc_segment_sums(flat_hbm, thr_hbm, out_hbm,
                     thr_v, buf0, buf1, acc_v, sem0, sem1):
    wid = lax.axis_index("s") * NCORES + lax.axis_index("c")
    base = wid * ROWS_PER_W

    pltpu.async_copy(flat_hbm.at[pl.ds(base, CHUNK)], buf0, sem0)
    pltpu.async_copy(flat_hbm.at[pl.ds(base + CHUNK, CHUNK)], buf1, sem1)

    pltpu.sync_copy(thr_hbm, thr_v)
    thr = thr_v[...]
    lane = lax.iota(jnp.int32, LANES)

    @plsc.parallel_loop(0, NSEG * VPR)
    def zero_body(i):
        acc_v[pl.ds(i * LANES, LANES)] = jnp.zeros((LANES,), jnp.float32)

    def seg_of(r):
        return jnp.sum((thr <= r).astype(jnp.int32))

    def process(buf, row0):
        seg0 = seg_of(row0)
        segN = seg_of(row0 + (CHUNK - 1))

        def fast():
            accbase = seg0 * D

            @plsc.parallel_loop(0, VPR, unroll=8)
            def fast_body(k):
                off = k * LANES
                col = pl.ds(off, LANES)
                vs = [buf[r, col] for r in range(CHUNK)]
                while len(vs) > 1:
                    vs = [vs[t] + vs[t + 1] for t in range(0, len(vs), 2)]
                plsc.addupdate(acc_v.at[pl.ds(accbase + off, LANES)], vs[0])

        def slow():
            def row_body(rl, _):
                rowbase = seg_of(row0 + rl) * D

                @plsc.parallel_loop(0, VPR, unroll=2)
                def inner(k):
                    v = buf[rl, pl.ds(k * LANES, LANES)]
                    plsc.addupdate(
                        acc_v.at[pl.ds(rowbase + k * LANES, LANES)], v)

                return _

            lax.fori_loop(0, CHUNK, row_body, None)

        lax.cond(seg0 == segN, fast, slow)

    def pair_body(i, _):
        c0 = 2 * i
        row0 = base + c0 * CHUNK

        pltpu.make_async_copy(
            flat_hbm.at[pl.ds(row0, CHUNK)], buf0, sem0).wait()
        process(buf0, row0)

        @pl.when(i < NPAIR - 1)
        def _start0():
            pltpu.async_copy(
                flat_hbm.at[pl.ds(row0 + 2 * CHUNK, CHUNK)], buf0, sem0)

        pltpu.make_async_copy(
            flat_hbm.at[pl.ds(row0 + CHUNK, CHUNK)], buf1, sem1).wait()
        process(buf1, row0 + CHUNK)

        @pl.when(i < NPAIR - 1)
        def _start1():
            pltpu.async_copy(
                flat_hbm.at[pl.ds(row0 + 3 * CHUNK, CHUNK)], buf1, sem1)

        return _

    lax.fori_loop(0, NPAIR, pair_body, None)
    pltpu.sync_copy(acc_v, out_hbm.at[wid])


def _combine_body(cu_ref, p_ref, o_ref):
    sums = jnp.sum(p_ref[...], axis=0)
    invs = []
    for b in range(NSEG):
        cnt = (cu_ref[b + 1] - cu_ref[b]).astype(jnp.float32)
        invs.append(jnp.full((1, 1), 1.0, jnp.float32) / jnp.maximum(cnt, 1.0))
    inv = jnp.concatenate(invs, axis=0)
    o_ref[...] = sums * inv


_combine = pl.pallas_call(
    _combine_body,
    out_shape=jax.ShapeDtypeStruct((NSEG, D), jnp.float32),
    in_specs=[
        pl.BlockSpec(memory_space=pltpu.SMEM),
        pl.BlockSpec(memory_space=pltpu.VMEM),
    ],
    out_specs=pl.BlockSpec(memory_space=pltpu.VMEM),
)


@jax.jit
def kernel(flat, cu_seqlens):
    thr = jnp.concatenate(
        [cu_seqlens[1:NSEG], jnp.full((1,), TOTAL, jnp.int32)]
    ).astype(jnp.int32)
    partials = _sc_segment_sums(flat, thr)
    partials = partials.reshape(NW, NSEG, D)
    return _combine(cu_seqlens, partials)


# speedup vs baseline: 1.2196x; 1.2196x over previous
import functools

import jax
import jax.numpy as jnp
from jax import lax
from jax.experimental import pallas as pl
from jax.experimental.pallas import tpu as pltpu
from jax.experimental.pallas import tpu_sc as plsc

NSEG = 16
D = 2048
TOTAL = 32768
LANES = 16
NCORES = 2
NSUB = 16
NW = NCORES * NSUB
TC_ROWS = 12288
SC_ROWS = TOTAL - TC_ROWS
ROWS_PER_W = SC_ROWS // NW
CHUNK = 16
NCHUNK = ROWS_PER_W // CHUNK
NPAIR = NCHUNK // 2
VPR = D // LANES
TCBLK = 1024

assert ROWS_PER_W % (2 * CHUNK) == 0
assert TC_ROWS % TCBLK == 0

_mesh = plsc.VectorSubcoreMesh(
    core_axis_name="c", subcore_axis_name="s",
    num_cores=NCORES, num_subcores=NSUB,
)


@functools.partial(
    pl.kernel,
    out_type=jax.ShapeDtypeStruct((NW, NSEG * D), jnp.float32),
    mesh=_mesh,
    compiler_params=pltpu.CompilerParams(needs_layout_passes=False),
    scratch_types=[
        pltpu.VMEM((LANES,), jnp.int32),
        pltpu.VMEM((CHUNK, D), jnp.float32),
        pltpu.VMEM((CHUNK, D), jnp.float32),
        pltpu.VMEM((NSEG * D,), jnp.float32),
        pltpu.SemaphoreType.DMA,
        pltpu.SemaphoreType.DMA,
    ],
)
def _sc_segment_sums(flat_hbm, thr_hbm, out_hbm,
                     thr_v, buf0, buf1, acc_v, sem0, sem1):
    wid = lax.axis_index("s") * NCORES + lax.axis_index("c")
    base = TC_ROWS + wid * ROWS_PER_W

    pltpu.async_copy(flat_hbm.at[pl.ds(base, CHUNK)], buf0, sem0)
    pltpu.async_copy(flat_hbm.at[pl.ds(base + CHUNK, CHUNK)], buf1, sem1)

    pltpu.sync_copy(thr_hbm, thr_v)
    thr = thr_v[...]

    @plsc.parallel_loop(0, NSEG * VPR)
    def zero_body(i):
        acc_v[pl.ds(i * LANES, LANES)] = jnp.zeros((LANES,), jnp.float32)

    def seg_of(r):
        return jnp.sum((thr <= r).astype(jnp.int32))

    def process(buf, row0):
        seg0 = seg_of(row0)
        segN = seg_of(row0 + (CHUNK - 1))

        def fast():
            accbase = seg0 * D

            @plsc.parallel_loop(0, VPR, unroll=4)
            def fast_body(k):
                off = k * LANES
                col = pl.ds(off, LANES)
                vs = [buf[r, col] for r in range(CHUNK)]
                while len(vs) > 1:
                    vs = [vs[t] + vs[t + 1] for t in range(0, len(vs), 2)]
                plsc.addupdate(acc_v.at[pl.ds(accbase + off, LANES)], vs[0])

        def slow():
            def row_body(rl, _):
                rowbase = seg_of(row0 + rl) * D

                @plsc.parallel_loop(0, VPR, unroll=2)
                def inner(k):
                    v = buf[rl, pl.ds(k * LANES, LANES)]
                    plsc.addupdate(
                        acc_v.at[pl.ds(rowbase + k * LANES, LANES)], v)

                return _

            lax.fori_loop(0, CHUNK, row_body, None)

        lax.cond(seg0 == segN, fast, slow)

    def pair_body(i, _):
        c0 = 2 * i
        row0 = base + c0 * CHUNK

        pltpu.make_async_copy(
            flat_hbm.at[pl.ds(row0, CHUNK)], buf0, sem0).wait()
        process(buf0, row0)

        @pl.when(i < NPAIR - 1)
        def _start0():
            pltpu.async_copy(
                flat_hbm.at[pl.ds(row0 + 2 * CHUNK, CHUNK)], buf0, sem0)

        pltpu.make_async_copy(
            flat_hbm.at[pl.ds(row0 + CHUNK, CHUNK)], buf1, sem1).wait()
        process(buf1, row0 + CHUNK)

        @pl.when(i < NPAIR - 1)
        def _start1():
            pltpu.async_copy(
                flat_hbm.at[pl.ds(row0 + 3 * CHUNK, CHUNK)], buf1, sem1)

        return _

    lax.fori_loop(0, NPAIR, pair_body, None)
    pltpu.sync_copy(acc_v, out_hbm.at[wid])


def _tc_body(cu_ref, x_ref, o_ref):
    i = pl.program_id(0)
    rows = lax.broadcasted_iota(jnp.int32, (NSEG, TCBLK), 1) + i * TCBLK
    lowb = jnp.concatenate(
        [jnp.full((1, 1), cu_ref[b], jnp.int32) for b in range(NSEG)], 0)
    upb = jnp.concatenate(
        [jnp.full((1, 1), cu_ref[b + 1], jnp.int32) for b in range(NSEG)], 0)
    onehot = ((rows >= lowb) & (rows < upb)).astype(jnp.float32)
    part = jnp.dot(onehot, x_ref[...], preferred_element_type=jnp.float32)

    @pl.when(i == 0)
    def _init():
        o_ref[...] = jnp.zeros_like(o_ref)

    o_ref[...] += part


_tc_segment_sums = pl.pallas_call(
    _tc_body,
    grid=(TC_ROWS // TCBLK,),
    out_shape=jax.ShapeDtypeStruct((NSEG, D), jnp.float32),
    in_specs=[
        pl.BlockSpec(memory_space=pltpu.SMEM),
        pl.BlockSpec((TCBLK, D), lambda i: (i, 0)),
    ],
    out_specs=pl.BlockSpec((NSEG, D), lambda i: (0, 0)),
    compiler_params=pltpu.CompilerParams(
        dimension_semantics=("arbitrary",)),
)


def _combine_body(cu_ref, p_ref, t_ref, o_ref):
    sums = jnp.sum(p_ref[...], axis=0) + t_ref[...]
    invs = []
    for b in range(NSEG):
        cnt = (cu_ref[b + 1] - cu_ref[b]).astype(jnp.float32)
        invs.append(jnp.full((1, 1), 1.0, jnp.float32) / jnp.maximum(cnt, 1.0))
    inv = jnp.concatenate(invs, axis=0)
    o_ref[...] = sums * inv


_combine = pl.pallas_call(
    _combine_body,
    out_shape=jax.ShapeDtypeStruct((NSEG, D), jnp.float32),
    in_specs=[
        pl.BlockSpec(memory_space=pltpu.SMEM),
        pl.BlockSpec(memory_space=pltpu.VMEM),
        pl.BlockSpec(memory_space=pltpu.VMEM),
    ],
    out_specs=pl.BlockSpec(memory_space=pltpu.VMEM),
)


@jax.jit
def kernel(flat, cu_seqlens):
    thr = jnp.concatenate(
        [cu_seqlens[1:NSEG], jnp.full((1,), TOTAL, jnp.int32)]
    ).astype(jnp.int32)
    partials = _sc_segment_sums(flat, thr)
    tc_part = _tc_segment_sums(cu_seqlens, flat)
    partials = partials.reshape(NW, NSEG, D)
    return _combine(cu_seqlens, partials, tc_part)


# speedup vs baseline: 1.2223x; 1.0022x over previous
import functools

import jax
import jax.numpy as jnp
from jax import lax
from jax.experimental import pallas as pl
from jax.experimental.pallas import tpu as pltpu
from jax.experimental.pallas import tpu_sc as plsc

NSEG = 16
D = 2048
TOTAL = 32768
LANES = 16
NCORES = 2
NSUB = 16
NW = NCORES * NSUB
TC_ROWS = 12288
SC_ROWS = TOTAL - TC_ROWS
ROWS_PER_W = SC_ROWS // NW
CHUNK = 16
NCHUNK = ROWS_PER_W // CHUNK
NPAIR = NCHUNK // 2
VPR = D // LANES
TCBLK = 1024

assert ROWS_PER_W % (2 * CHUNK) == 0
assert TC_ROWS % TCBLK == 0

_mesh = plsc.VectorSubcoreMesh(
    core_axis_name="c", subcore_axis_name="s",
    num_cores=NCORES, num_subcores=NSUB,
)


@functools.partial(
    pl.kernel,
    out_type=jax.ShapeDtypeStruct((NW, NSEG * D), jnp.float32),
    mesh=_mesh,
    compiler_params=pltpu.CompilerParams(needs_layout_passes=False,
                                         skip_device_barrier=True),
    scratch_types=[
        pltpu.VMEM((LANES,), jnp.int32),
        pltpu.VMEM((CHUNK, D), jnp.float32),
        pltpu.VMEM((CHUNK, D), jnp.float32),
        pltpu.VMEM((NSEG * D,), jnp.float32),
        pltpu.SemaphoreType.DMA,
        pltpu.SemaphoreType.DMA,
    ],
)
def _sc_segment_sums(flat_hbm, thr_hbm, out_hbm,
                     thr_v, buf0, buf1, acc_v, sem0, sem1):
    wid = lax.axis_index("s") * NCORES + lax.axis_index("c")
    base = TC_ROWS + wid * ROWS_PER_W

    pltpu.async_copy(flat_hbm.at[pl.ds(base, CHUNK)], buf0, sem0)
    pltpu.async_copy(flat_hbm.at[pl.ds(base + CHUNK, CHUNK)], buf1, sem1)

    pltpu.sync_copy(thr_hbm, thr_v)
    thr = thr_v[...]

    @plsc.parallel_loop(0, NSEG * VPR)
    def zero_body(i):
        acc_v[pl.ds(i * LANES, LANES)] = jnp.zeros((LANES,), jnp.float32)

    def seg_of(r):
        return jnp.sum((thr <= r).astype(jnp.int32))

    def process(buf, row0):
        seg0 = seg_of(row0)
        segN = seg_of(row0 + (CHUNK - 1))

        def fast():
            accbase = seg0 * D

            @plsc.parallel_loop(0, VPR, unroll=4)
            def fast_body(k):
                off = k * LANES
                col = pl.ds(off, LANES)
                vs = [buf[r, col] for r in range(CHUNK)]
                while len(vs) > 1:
                    vs = [vs[t] + vs[t + 1] for t in range(0, len(vs), 2)]
                plsc.addupdate(acc_v.at[pl.ds(accbase + off, LANES)], vs[0])

        def slow():
            def row_body(rl, _):
                rowbase = seg_of(row0 + rl) * D

                @plsc.parallel_loop(0, VPR, unroll=2)
                def inner(k):
                    v = buf[rl, pl.ds(k * LANES, LANES)]
                    plsc.addupdate(
                        acc_v.at[pl.ds(rowbase + k * LANES, LANES)], v)

                return _

            lax.fori_loop(0, CHUNK, row_body, None)

        lax.cond(seg0 == segN, fast, slow)

    def pair_body(i, _):
        c0 = 2 * i
        row0 = base + c0 * CHUNK

        pltpu.make_async_copy(
            flat_hbm.at[pl.ds(row0, CHUNK)], buf0, sem0).wait()
        process(buf0, row0)

        @pl.when(i < NPAIR - 1)
        def _start0():
            pltpu.async_copy(
                flat_hbm.at[pl.ds(row0 + 2 * CHUNK, CHUNK)], buf0, sem0)

        pltpu.make_async_copy(
            flat_hbm.at[pl.ds(row0 + CHUNK, CHUNK)], buf1, sem1).wait()
        process(buf1, row0 + CHUNK)

        @pl.when(i < NPAIR - 1)
        def _start1():
            pltpu.async_copy(
                flat_hbm.at[pl.ds(row0 + 3 * CHUNK, CHUNK)], buf1, sem1)

        return _

    lax.fori_loop(0, NPAIR, pair_body, None)
    pltpu.sync_copy(acc_v, out_hbm.at[wid])


def _tc_body(cu_ref, x_ref, o_ref):
    i = pl.program_id(0)
    rows = lax.broadcasted_iota(jnp.int32, (NSEG, TCBLK), 1) + i * TCBLK
    lowb = jnp.concatenate(
        [jnp.full((1, 1), cu_ref[b], jnp.int32) for b in range(NSEG)], 0)
    upb = jnp.concatenate(
        [jnp.full((1, 1), cu_ref[b + 1], jnp.int32) for b in range(NSEG)], 0)
    onehot = ((rows >= lowb) & (rows < upb)).astype(jnp.float32)
    part = jnp.dot(onehot, x_ref[...], preferred_element_type=jnp.float32)

    @pl.when(i == 0)
    def _init():
        o_ref[...] = jnp.zeros_like(o_ref)

    o_ref[...] += part


_tc_segment_sums = pl.pallas_call(
    _tc_body,
    grid=(TC_ROWS // TCBLK,),
    out_shape=jax.ShapeDtypeStruct((NSEG, D), jnp.float32),
    in_specs=[
        pl.BlockSpec(memory_space=pltpu.SMEM),
        pl.BlockSpec((TCBLK, D), lambda i: (i, 0)),
    ],
    out_specs=pl.BlockSpec((NSEG, D), lambda i: (0, 0)),
    compiler_params=pltpu.CompilerParams(
        dimension_semantics=("arbitrary",)),
)


def _combine_body(cu_ref, p_ref, t_ref, o_ref):
    sums = jnp.sum(p_ref[...], axis=0) + t_ref[...]
    invs = []
    for b in range(NSEG):
        cnt = (cu_ref[b + 1] - cu_ref[b]).astype(jnp.float32)
        invs.append(jnp.full((1, 1), 1.0, jnp.float32) / jnp.maximum(cnt, 1.0))
    inv = jnp.concatenate(invs, axis=0)
    o_ref[...] = sums * inv


_combine = pl.pallas_call(
    _combine_body,
    out_shape=jax.ShapeDtypeStruct((NSEG, D), jnp.float32),
    in_specs=[
        pl.BlockSpec(memory_space=pltpu.SMEM),
        pl.BlockSpec(memory_space=pltpu.VMEM),
        pl.BlockSpec(memory_space=pltpu.VMEM),
    ],
    out_specs=pl.BlockSpec(memory_space=pltpu.VMEM),
)


@jax.jit
def kernel(flat, cu_seqlens):
    thr = jnp.concatenate(
        [cu_seqlens[1:NSEG], jnp.full((1,), TOTAL, jnp.int32)]
    ).astype(jnp.int32)
    tc_part = _tc_segment_sums(cu_seqlens, flat)
    partials = _sc_segment_sums(flat, thr)
    partials = partials.reshape(NW, NSEG, D)
    return _combine(cu_seqlens, partials, tc_part)


# speedup vs baseline: 1.3075x; 1.0697x over previous
import functools

import jax
import jax.numpy as jnp
from jax import lax
from jax.experimental import pallas as pl
from jax.experimental.pallas import tpu as pltpu
from jax.experimental.pallas import tpu_sc as plsc

NSEG = 16
D = 2048
TOTAL = 32768
LANES = 16
NCORES = 2
NSUB = 16
NW = NCORES * NSUB
TC_ROWS = 16384
SC_ROWS = TOTAL - TC_ROWS
ROWS_PER_W = SC_ROWS // NW
CHUNK = 16
NCHUNK = ROWS_PER_W // CHUNK
NPAIR = NCHUNK // 2
VPR = D // LANES
TCBLK = 1024

assert ROWS_PER_W % (2 * CHUNK) == 0
assert TC_ROWS % TCBLK == 0

_mesh = plsc.VectorSubcoreMesh(
    core_axis_name="c", subcore_axis_name="s",
    num_cores=NCORES, num_subcores=NSUB,
)


@functools.partial(
    pl.kernel,
    out_type=jax.ShapeDtypeStruct((NW, NSEG * D), jnp.float32),
    mesh=_mesh,
    compiler_params=pltpu.CompilerParams(needs_layout_passes=False,
                                         skip_device_barrier=True),
    scratch_types=[
        pltpu.VMEM((LANES,), jnp.int32),
        pltpu.VMEM((CHUNK, D), jnp.float32),
        pltpu.VMEM((CHUNK, D), jnp.float32),
        pltpu.VMEM((NSEG * D,), jnp.float32),
        pltpu.SemaphoreType.DMA,
        pltpu.SemaphoreType.DMA,
    ],
)
def _sc_segment_sums(flat_hbm, thr_hbm, out_hbm,
                     thr_v, buf0, buf1, acc_v, sem0, sem1):
    wid = lax.axis_index("s") * NCORES + lax.axis_index("c")
    base = TC_ROWS + wid * ROWS_PER_W

    pltpu.async_copy(flat_hbm.at[pl.ds(base, CHUNK)], buf0, sem0)
    pltpu.async_copy(flat_hbm.at[pl.ds(base + CHUNK, CHUNK)], buf1, sem1)

    pltpu.sync_copy(thr_hbm, thr_v)
    thr = thr_v[...]

    @plsc.parallel_loop(0, NSEG * VPR)
    def zero_body(i):
        acc_v[pl.ds(i * LANES, LANES)] = jnp.zeros((LANES,), jnp.float32)

    def seg_of(r):
        return jnp.sum((thr <= r).astype(jnp.int32))

    def process(buf, row0):
        seg0 = seg_of(row0)
        segN = seg_of(row0 + (CHUNK - 1))

        def fast():
            accbase = seg0 * D

            @plsc.parallel_loop(0, VPR, unroll=4)
            def fast_body(k):
                off = k * LANES
                col = pl.ds(off, LANES)
                vs = [buf[r, col] for r in range(CHUNK)]
                while len(vs) > 1:
                    vs = [vs[t] + vs[t + 1] for t in range(0, len(vs), 2)]
                plsc.addupdate(acc_v.at[pl.ds(accbase + off, LANES)], vs[0])

        def slow():
            def row_body(rl, _):
                rowbase = seg_of(row0 + rl) * D

                @plsc.parallel_loop(0, VPR, unroll=2)
                def inner(k):
                    v = buf[rl, pl.ds(k * LANES, LANES)]
                    plsc.addupdate(
                        acc_v.at[pl.ds(rowbase + k * LANES, LANES)], v)

                return _

            lax.fori_loop(0, CHUNK, row_body, None)

        lax.cond(seg0 == segN, fast, slow)

    def pair_body(i, _):
        c0 = 2 * i
        row0 = base + c0 * CHUNK

        pltpu.make_async_copy(
            flat_hbm.at[pl.ds(row0, CHUNK)], buf0, sem0).wait()
        process(buf0, row0)

        @pl.when(i < NPAIR - 1)
        def _start0():
            pltpu.async_copy(
                flat_hbm.at[pl.ds(row0 + 2 * CHUNK, CHUNK)], buf0, sem0)

        pltpu.make_async_copy(
            flat_hbm.at[pl.ds(row0 + CHUNK, CHUNK)], buf1, sem1).wait()
        process(buf1, row0 + CHUNK)

        @pl.when(i < NPAIR - 1)
        def _start1():
            pltpu.async_copy(
                flat_hbm.at[pl.ds(row0 + 3 * CHUNK, CHUNK)], buf1, sem1)

        return _

    lax.fori_loop(0, NPAIR, pair_body, None)
    pltpu.sync_copy(acc_v, out_hbm.at[wid])


def _tc_body(cu_ref, x_ref, o_ref):
    i = pl.program_id(0)
    rows = lax.broadcasted_iota(jnp.int32, (NSEG, TCBLK), 1) + i * TCBLK
    lowb = jnp.concatenate(
        [jnp.full((1, 1), cu_ref[b], jnp.int32) for b in range(NSEG)], 0)
    upb = jnp.concatenate(
        [jnp.full((1, 1), cu_ref[b + 1], jnp.int32) for b in range(NSEG)], 0)
    onehot = ((rows >= lowb) & (rows < upb)).astype(jnp.float32)
    part = jnp.dot(onehot, x_ref[...], preferred_element_type=jnp.float32)

    @pl.when(i == 0)
    def _init():
        o_ref[...] = jnp.zeros_like(o_ref)

    o_ref[...] += part


_tc_segment_sums = pl.pallas_call(
    _tc_body,
    grid=(TC_ROWS // TCBLK,),
    out_shape=jax.ShapeDtypeStruct((NSEG, D), jnp.float32),
    in_specs=[
        pl.BlockSpec(memory_space=pltpu.SMEM),
        pl.BlockSpec((TCBLK, D), lambda i: (i, 0)),
    ],
    out_specs=pl.BlockSpec((NSEG, D), lambda i: (0, 0)),
    compiler_params=pltpu.CompilerParams(
        dimension_semantics=("arbitrary",)),
)


def _combine_body(cu_ref, p_ref, t_ref, o_ref):
    sums = jnp.sum(p_ref[...], axis=0) + t_ref[...]
    invs = []
    for b in range(NSEG):
        cnt = (cu_ref[b + 1] - cu_ref[b]).astype(jnp.float32)
        invs.append(jnp.full((1, 1), 1.0, jnp.float32) / jnp.maximum(cnt, 1.0))
    inv = jnp.concatenate(invs, axis=0)
    o_ref[...] = sums * inv


_combine = pl.pallas_call(
    _combine_body,
    out_shape=jax.ShapeDtypeStruct((NSEG, D), jnp.float32),
    in_specs=[
        pl.BlockSpec(memory_space=pltpu.SMEM),
        pl.BlockSpec(memory_space=pltpu.VMEM),
        pl.BlockSpec(memory_space=pltpu.VMEM),
    ],
    out_specs=pl.BlockSpec(memory_space=pltpu.VMEM),
)


@jax.jit
def kernel(flat, cu_seqlens):
    thr = jnp.concatenate(
        [cu_seqlens[1:NSEG], jnp.full((1,), TOTAL, jnp.int32)]
    ).astype(jnp.int32)
    tc_part = _tc_segment_sums(cu_seqlens, flat)
    partials = _sc_segment_sums(flat, thr)
    partials = partials.reshape(NW, NSEG, D)
    return _combine(cu_seqlens, partials, tc_part)


# speedup vs baseline: 1.3076x; 1.0001x over previous
import functools

import jax
import jax.numpy as jnp
from jax import lax
from jax.experimental import pallas as pl
from jax.experimental.pallas import tpu as pltpu
from jax.experimental.pallas import tpu_sc as plsc

NSEG = 16
D = 2048
TOTAL = 32768
LANES = 16
NCORES = 2
NSUB = 16
NW = NCORES * NSUB
TC_ROWS = 16384
SC_ROWS = TOTAL - TC_ROWS
ROWS_PER_W = SC_ROWS // NW
CHUNK = 16
NCHUNK = ROWS_PER_W // CHUNK
NPAIR = NCHUNK // 2
VPR = D // LANES
TCBLK = 2048

assert ROWS_PER_W % (2 * CHUNK) == 0
assert TC_ROWS % TCBLK == 0

_mesh = plsc.VectorSubcoreMesh(
    core_axis_name="c", subcore_axis_name="s",
    num_cores=NCORES, num_subcores=NSUB,
)


@functools.partial(
    pl.kernel,
    out_type=jax.ShapeDtypeStruct((NW, NSEG * D), jnp.float32),
    mesh=_mesh,
    compiler_params=pltpu.CompilerParams(needs_layout_passes=False,
                                         skip_device_barrier=True),
    scratch_types=[
        pltpu.VMEM((LANES,), jnp.int32),
        pltpu.VMEM((CHUNK, D), jnp.float32),
        pltpu.VMEM((CHUNK, D), jnp.float32),
        pltpu.VMEM((NSEG * D,), jnp.float32),
        pltpu.SemaphoreType.DMA,
        pltpu.SemaphoreType.DMA,
    ],
)
def _sc_segment_sums(flat_hbm, thr_hbm, out_hbm,
                     thr_v, buf0, buf1, acc_v, sem0, sem1):
    wid = lax.axis_index("s") * NCORES + lax.axis_index("c")
    base = TC_ROWS + wid * ROWS_PER_W

    pltpu.async_copy(flat_hbm.at[pl.ds(base, CHUNK)], buf0, sem0)
    pltpu.async_copy(flat_hbm.at[pl.ds(base + CHUNK, CHUNK)], buf1, sem1)

    pltpu.sync_copy(thr_hbm, thr_v)
    thr = thr_v[...]

    @plsc.parallel_loop(0, NSEG * VPR)
    def zero_body(i):
        acc_v[pl.ds(i * LANES, LANES)] = jnp.zeros((LANES,), jnp.float32)

    def seg_of(r):
        return jnp.sum((thr <= r).astype(jnp.int32))

    def process(buf, row0):
        seg0 = seg_of(row0)
        segN = seg_of(row0 + (CHUNK - 1))

        def fast():
            accbase = seg0 * D

            @plsc.parallel_loop(0, VPR, unroll=4)
            def fast_body(k):
                off = k * LANES
                col = pl.ds(off, LANES)
                vs = [buf[r, col] for r in range(CHUNK)]
                while len(vs) > 1:
                    vs = [vs[t] + vs[t + 1] for t in range(0, len(vs), 2)]
                plsc.addupdate(acc_v.at[pl.ds(accbase + off, LANES)], vs[0])

        def slow():
            def row_body(rl, _):
                rowbase = seg_of(row0 + rl) * D

                @plsc.parallel_loop(0, VPR, unroll=2)
                def inner(k):
                    v = buf[rl, pl.ds(k * LANES, LANES)]
                    plsc.addupdate(
                        acc_v.at[pl.ds(rowbase + k * LANES, LANES)], v)

                return _

            lax.fori_loop(0, CHUNK, row_body, None)

        lax.cond(seg0 == segN, fast, slow)

    def pair_body(i, _):
        c0 = 2 * i
        row0 = base + c0 * CHUNK

        pltpu.make_async_copy(
            flat_hbm.at[pl.ds(row0, CHUNK)], buf0, sem0).wait()
        process(buf0, row0)

        @pl.when(i < NPAIR - 1)
        def _start0():
            pltpu.async_copy(
                flat_hbm.at[pl.ds(row0 + 2 * CHUNK, CHUNK)], buf0, sem0)

        pltpu.make_async_copy(
            flat_hbm.at[pl.ds(row0 + CHUNK, CHUNK)], buf1, sem1).wait()
        process(buf1, row0 + CHUNK)

        @pl.when(i < NPAIR - 1)
        def _start1():
            pltpu.async_copy(
                flat_hbm.at[pl.ds(row0 + 3 * CHUNK, CHUNK)], buf1, sem1)

        return _

    lax.fori_loop(0, NPAIR, pair_body, None)
    pltpu.sync_copy(acc_v, out_hbm.at[wid])


def _tc_body(cu_ref, x_ref, o_ref):
    i = pl.program_id(0)
    rows = lax.broadcasted_iota(jnp.int32, (NSEG, TCBLK), 1) + i * TCBLK
    lowb = jnp.concatenate(
        [jnp.full((1, 1), cu_ref[b], jnp.int32) for b in range(NSEG)], 0)
    upb = jnp.concatenate(
        [jnp.full((1, 1), cu_ref[b + 1], jnp.int32) for b in range(NSEG)], 0)
    onehot = ((rows >= lowb) & (rows < upb)).astype(jnp.float32)
    part = jnp.dot(onehot, x_ref[...], preferred_element_type=jnp.float32)

    @pl.when(i == 0)
    def _init():
        o_ref[...] = jnp.zeros_like(o_ref)

    o_ref[...] += part


_tc_segment_sums = pl.pallas_call(
    _tc_body,
    grid=(TC_ROWS // TCBLK,),
    out_shape=jax.ShapeDtypeStruct((NSEG, D), jnp.float32),
    in_specs=[
        pl.BlockSpec(memory_space=pltpu.SMEM),
        pl.BlockSpec((TCBLK, D), lambda i: (i, 0)),
    ],
    out_specs=pl.BlockSpec((NSEG, D), lambda i: (0, 0)),
    compiler_params=pltpu.CompilerParams(
        dimension_semantics=("arbitrary",)),
)


def _combine_body(cu_ref, p_ref, t_ref, o_ref):
    sums = jnp.sum(p_ref[...], axis=0) + t_ref[...]
    invs = []
    for b in range(NSEG):
        cnt = (cu_ref[b + 1] - cu_ref[b]).astype(jnp.float32)
        invs.append(jnp.full((1, 1), 1.0, jnp.float32) / jnp.maximum(cnt, 1.0))
    inv = jnp.concatenate(invs, axis=0)
    o_ref[...] = sums * inv


_combine = pl.pallas_call(
    _combine_body,
    out_shape=jax.ShapeDtypeStruct((NSEG, D), jnp.float32),
    in_specs=[
        pl.BlockSpec(memory_space=pltpu.SMEM),
        pl.BlockSpec(memory_space=pltpu.VMEM),
        pl.BlockSpec(memory_space=pltpu.VMEM),
    ],
    out_specs=pl.BlockSpec(memory_space=pltpu.VMEM),
)


@jax.jit
def kernel(flat, cu_seqlens):
    thr = jnp.concatenate(
        [cu_seqlens[1:NSEG], jnp.full((1,), TOTAL, jnp.int32)]
    ).astype(jnp.int32)
    tc_part = _tc_segment_sums(cu_seqlens, flat)
    partials = _sc_segment_sums(flat, thr)
    partials = partials.reshape(NW, NSEG, D)
    return _combine(cu_seqlens, partials, tc_part)
